# baseline (speedup 1.0000x reference)
import functools

import jax
import jax.numpy as jnp
from jax import lax
from jax.experimental import pallas as pl
from jax.experimental.pallas import tpu as pltpu, tpu_sc as plsc

_L = 16
_BS = 2


def _tc_body(x_ref, t_ref, o_ref):
    o_ref[...] = x_ref[...] + t_ref[...][None]


def _tc_part(xf, pos_embedding, nb):
    B, S, d = xf.shape
    R = 512
    return pl.pallas_call(
        _tc_body,
        grid=(S // R, nb),
        in_specs=[
            pl.BlockSpec((1, R, d), lambda s, b: (b, s, 0)),
            pl.BlockSpec((R, d), lambda s, b: (s, 0)),
        ],
        out_specs=pl.BlockSpec((1, R, d), lambda s, b: (b, s, 0)),
        out_shape=jax.ShapeDtypeStruct((nb, S, d), xf.dtype),
    )(xf, pos_embedding)


def _make_sc_kernel(B, b_lo, S, d, NC, NS):
    nb = B - b_lo
    NW = NC * NS
    rows_per_w = S // NW
    CH = 32
    n_chunks = rows_per_w // CH
    n_vregs = d // _L
    mesh = plsc.VectorSubcoreMesh(core_axis_name="c", subcore_axis_name="s")

    @functools.partial(
        pl.kernel,
        out_type=jax.ShapeDtypeStruct((nb, S, d), jnp.float32),
        mesh=mesh,
        scratch_types=[
            pltpu.VMEM((CH, d), jnp.float32),
            pltpu.VMEM((CH, d), jnp.float32),
            pltpu.VMEM((CH, d), jnp.float32),
            pltpu.VMEM((CH, d), jnp.float32),
            pltpu.VMEM((CH, d), jnp.float32),
            pltpu.SemaphoreType.DMA,
            pltpu.SemaphoreType.DMA,
            pltpu.SemaphoreType.DMA,
            pltpu.SemaphoreType.DMA,
            pltpu.SemaphoreType.DMA,
            pltpu.SemaphoreType.DMA,
            pltpu.SemaphoreType.DMA,
            pltpu.SemaphoreType.DMA,
        ],
    )
    def sc_kernel(x_hbm, tbl_hbm, out_hbm, tbl_v0, tbl_v1, buf_v0, buf_v1,
                  buf_v2, sx0, sx1, sx2, so0, so1, so2, st0, st1):
        tbl_v = (tbl_v0, tbl_v1)
        buf_v = (buf_v0, buf_v1, buf_v2)
        sx = (sx0, sx1, sx2)
        so = (so0, so1, so2)
        st = (st0, st1)
        wid = lax.axis_index("s") * NC + lax.axis_index("c")
        base = wid * rows_per_w

        items = [(c, b) for c in range(n_chunks) for b in range(b_lo, B)]

        def rows(c):
            return pl.ds(base + c * CH, CH)

        tbl_cp = {0: pltpu.async_copy(tbl_hbm.at[rows(0)], tbl_v[0], st[0])}
        x_cp = {0: pltpu.async_copy(
            x_hbm.at[items[0][1], rows(0)], buf_v[0], sx[0])}
        out_cp = {}

        for k, (c, b) in enumerate(items):
            ib = k % 3
            if k + 1 < len(items):
                c2, b2 = items[k + 1]
                nb3 = (k + 1) % 3
                if nb3 in out_cp:
                    out_cp.pop(nb3).wait()
                x_cp[k + 1] = pltpu.async_copy(
                    x_hbm.at[b2, rows(c2)], buf_v[nb3], sx[nb3])
                if b2 == b_lo and c2 not in tbl_cp:
                    tbl_cp[c2] = pltpu.async_copy(
                        tbl_hbm.at[rows(c2)], tbl_v[c2 % 2], st[c2 % 2])
            if b == b_lo:
                tbl_cp[c].wait()
            x_cp.pop(k).wait()
            buf = buf_v[ib]
            tbl = tbl_v[c % 2]

            @plsc.parallel_loop(0, CH, 1)
            def _(r):
                @plsc.parallel_loop(0, n_vregs, 1, unroll=8)
                def _(j):
                    sl = pl.ds(j * _L, _L)
                    buf[r, sl] = buf[r, sl] + tbl[r, sl]

            out_cp[ib] = pltpu.async_copy(
                buf, out_hbm.at[b - b_lo, rows(c)], so[ib])

        for cp in out_cp.values():
            cp.wait()

    return sc_kernel


def kernel(x, pos_embedding):
    B, D1, D2, d = x.shape
    S = D1 * D2
    xf = x.reshape(B, S, d)
    info = plsc.get_sparse_core_info()
    sc = _make_sc_kernel(B, _BS, S, d, info.num_cores, info.num_subcores)
    out_sc = sc(xf, pos_embedding)
    out_tc = _tc_part(xf, pos_embedding, _BS)
    out = jnp.concatenate([out_tc, out_sc], axis=0)
    return out.reshape(B, D1, D2, d)


# speedup vs baseline: 1.5007x; 1.5007x over previous
import functools

import jax
import jax.numpy as jnp
from jax import lax
from jax.experimental import pallas as pl
from jax.experimental.pallas import tpu as pltpu, tpu_sc as plsc

_L = 16


def _make_sc_kernel(B, S, d, NC, NS):
    NW = NC * NS
    rows_per_w = S // NW
    CH = 32
    n_chunks = rows_per_w // CH
    n_vregs = d // _L
    mesh = plsc.VectorSubcoreMesh(core_axis_name="c", subcore_axis_name="s")

    @functools.partial(
        pl.kernel,
        out_type=jax.ShapeDtypeStruct((B, S, d), jnp.float32),
        mesh=mesh,
        scratch_types=[
            pltpu.VMEM((CH, d), jnp.float32),
            pltpu.VMEM((CH, d), jnp.float32),
            pltpu.VMEM((CH, d), jnp.float32),
            pltpu.VMEM((CH, d), jnp.float32),
            pltpu.VMEM((CH, d), jnp.float32),
            pltpu.SemaphoreType.DMA,
            pltpu.SemaphoreType.DMA,
            pltpu.SemaphoreType.DMA,
            pltpu.SemaphoreType.DMA,
            pltpu.SemaphoreType.DMA,
            pltpu.SemaphoreType.DMA,
            pltpu.SemaphoreType.DMA,
            pltpu.SemaphoreType.DMA,
        ],
    )
    def sc_kernel(x_hbm, tbl_hbm, out_hbm, tbl_v0, tbl_v1, buf_v0, buf_v1,
                  buf_v2, sx0, sx1, sx2, so0, so1, so2, st0, st1):
        tbl_v = (tbl_v0, tbl_v1)
        buf_v = (buf_v0, buf_v1, buf_v2)
        sx = (sx0, sx1, sx2)
        so = (so0, so1, so2)
        st = (st0, st1)
        wid = lax.axis_index("s") * NC + lax.axis_index("c")
        base = wid * rows_per_w

        items = [(c, b) for c in range(n_chunks) for b in range(B)]
        n_items = len(items)

        def rows(c):
            return pl.ds(base + c * CH, CH)

        def start_x(k):
            c, b = items[k]
            return pltpu.async_copy(x_hbm.at[b, rows(c)], buf_v[k % 3],
                                    sx[k % 3])

        def start_tbl(c):
            return pltpu.async_copy(tbl_hbm.at[rows(c)], tbl_v[c % 2],
                                    st[c % 2])

        tbl_cp = {0: start_tbl(0)}
        if n_chunks > 1:
            tbl_cp[1] = start_tbl(1)
        x_cp = {0: start_x(0)}
        if n_items > 1:
            x_cp[1] = start_x(1)
        out_cp = {}

        for k, (c, b) in enumerate(items):
            if b == 0:
                tbl_cp[c].wait()
            x_cp.pop(k).wait()
            buf = buf_v[k % 3]
            tbl = tbl_v[c % 2]

            @plsc.parallel_loop(0, CH, 1)
            def _(r):
                @plsc.parallel_loop(0, n_vregs, 1, unroll=8)
                def _(j):
                    sl = pl.ds(j * _L, _L)
                    buf[r, sl] = buf[r, sl] + tbl[r, sl]

            out_cp[k % 3] = pltpu.async_copy(
                buf, out_hbm.at[b, rows(c)], so[k % 3])
            if k + 2 < n_items:
                nb = (k + 2) % 3
                if nb in out_cp:
                    out_cp.pop(nb).wait()
                x_cp[k + 2] = start_x(k + 2)
                c2 = items[k + 2][0]
                if c2 not in tbl_cp and c2 == c + 1:
                    tbl_cp[c2] = start_tbl(c2)

        for cp in out_cp.values():
            cp.wait()

    return sc_kernel


def kernel(x, pos_embedding):
    B, D1, D2, d = x.shape
    S = D1 * D2
    info = plsc.get_sparse_core_info()
    sc = _make_sc_kernel(B, S, d, info.num_cores, info.num_subcores)
    out = sc(x.reshape(B, S, d), pos_embedding)
    return out.reshape(B, D1, D2, d)
